# baseline (speedup 1.0000x reference)
import functools

import jax
import jax.numpy as jnp
from jax import lax
from jax.experimental import pallas as pl
from jax.experimental.pallas import tpu as pltpu
from jax.experimental.pallas import tpu_sc as plsc

_N = 10000
_E = 160000
_D = 256
_H = 128
_NC = 2
_NS = 16
_K = 64
_EP = _E // _NS
_NCH = 160
_EPP = _NCH * _K
_NPAD = 10240
_RPZ = _NPAD // _NS
_RPA = 632
_RPT = _N - 15 * _RPA
_ZROW = 0
_CC = _NCH // _NC
_GW = 16
_GWC = 8
_NB_SEG = 5
_DLY = 3


def _make_sc_segsum():
  mesh = plsc.VectorSubcoreMesh(core_axis_name="c", subcore_axis_name="s",
                                num_cores=_NC, num_subcores=_NS)
  out_type = jax.ShapeDtypeStruct((_NC * _N, _H), jnp.float32)
  scratch = [
      pltpu.VMEM((_GW, _K), jnp.int32),
      pltpu.VMEM((_GW, _K), jnp.int32),
  ] + [pltpu.VMEM((_K, _H), jnp.float32) for _ in range(_NB_SEG)] + [
      pltpu.VMEM_SHARED((_NPAD, _H), jnp.float32),
  ] + [pltpu.SemaphoreType.DMA for _ in range(2 * _NB_SEG)]

  def body(xs, sidx, didx, zf, agg_out, sidx_v, didx_v, *rest):
    bufs = rest[:_NB_SEG]
    agg_sp = rest[_NB_SEG]
    gsems = rest[_NB_SEG + 1:2 * _NB_SEG + 1]
    ssems = rest[2 * _NB_SEG + 1:]
    c = lax.axis_index("c")
    s = lax.axis_index("s")

    pltpu.sync_copy(zf.at[pl.ds(s * _RPZ, _RPZ)],
                    agg_sp.at[pl.ds(s * _RPZ, _RPZ)])
    plsc.subcore_barrier()

    def group(g, carry):
      pltpu.sync_copy(sidx.at[c, s, pl.ds(g * _GW, _GW)], sidx_v)
      pltpu.sync_copy(didx.at[s, pl.ds(g * _GW, _GW)], didx_v)
      gd = {}
      sd = {}

      def scat(j):
        gd[j].wait()
        b = j % _NB_SEG
        sd[j] = pltpu.async_copy(bufs[b], agg_sp.at[didx_v.at[j]],
                                 ssems[b], add=True)

      for j in range(_GW):
        b = j % _NB_SEG
        if j >= _NB_SEG:
          sd[j - _NB_SEG].wait()
        gd[j] = pltpu.async_copy(xs.at[sidx_v.at[j]], bufs[b], gsems[b])
        if j >= _DLY:
          scat(j - _DLY)
      for j in range(_GW - _DLY, _GW):
        scat(j)
      for j in range(_GW - _NB_SEG, _GW):
        sd[j].wait()
      return carry
    lax.fori_loop(0, _NCH // _GW, group, 0)
    plsc.subcore_barrier()

    @pl.when(s < _NS - 1)
    def _():
      pltpu.sync_copy(agg_sp.at[pl.ds(s * _RPA, _RPA)],
                      agg_out.at[pl.ds(c * _N + s * _RPA, _RPA)])

    @pl.when(s == _NS - 1)
    def _():
      base = 15 * _RPA
      pltpu.sync_copy(agg_sp.at[pl.ds(base, _RPT)],
                      agg_out.at[pl.ds(c * _N + base, _RPT)])

  return pl.kernel(body, out_type=out_type, mesh=mesh, scratch_types=scratch)


def _make_sc_count():
  mesh = plsc.VectorSubcoreMesh(core_axis_name="c", subcore_axis_name="s",
                                num_cores=_NC, num_subcores=_NS)
  out_type = jax.ShapeDtypeStruct((_NC * _N, _H), jnp.float32)
  scratch = [
      pltpu.VMEM((_GWC, _K), jnp.int32),
      pltpu.VMEM((_K, _H), jnp.float32),
      pltpu.VMEM_SHARED((_NPAD, _H), jnp.float32),
      pltpu.SemaphoreType.DMA,
  ]

  def body(didx_cnt, zc, ones_h, cnt_out, didx_v, ones_v, cnt_sp, semc):
    c = lax.axis_index("c")
    s = lax.axis_index("s")
    pltpu.sync_copy(ones_h, ones_v)
    pltpu.sync_copy(zc.at[pl.ds(s * _RPZ, _RPZ)],
                    cnt_sp.at[pl.ds(s * _RPZ, _RPZ)])
    plsc.subcore_barrier()

    def group(g, carry):
      pltpu.sync_copy(didx_cnt.at[c, s, pl.ds(g * _GWC, _GWC)], didx_v)
      ds = []
      for j in range(_GWC):
        ds.append(pltpu.async_copy(ones_v, cnt_sp.at[didx_v.at[j]], semc,
                                   add=True))
      for dd in ds:
        dd.wait()
      return carry
    lax.fori_loop(0, _CC // _GWC, group, 0)
    plsc.subcore_barrier()

    @pl.when(s < _NS - 1)
    def _():
      pltpu.sync_copy(cnt_sp.at[pl.ds(s * _RPA, _RPA)],
                      cnt_out.at[pl.ds(c * _N + s * _RPA, _RPA)])

    @pl.when(s == _NS - 1)
    def _():
      base = 15 * _RPA
      pltpu.sync_copy(cnt_sp.at[pl.ds(base, _RPT)],
                      cnt_out.at[pl.ds(c * _N + base, _RPT)])

  return pl.kernel(body, out_type=out_type, mesh=mesh, scratch_types=scratch)


_BR = 1000
_NB = _N // _BR


def _layer_tc_body(agg0, agg1, cnt0, cnt1, x0, x1, wl, wr, bl, out):
  cnt = cnt0[:, 0:1] + cnt1[:, 0:1]
  rcnt = 1.0 / jnp.maximum(cnt, 1.0)
  m0 = agg0[...] * rcnt
  m1 = agg1[...] * rcnt
  acc = jnp.dot(m0, wl[0:_H, :], preferred_element_type=jnp.float32)
  acc += jnp.dot(m1, wl[_H:_D, :], preferred_element_type=jnp.float32)
  acc += jnp.dot(x0[...], wr[0:_H, :], preferred_element_type=jnp.float32)
  acc += jnp.dot(x1[...], wr[_H:_D, :], preferred_element_type=jnp.float32)
  out[...] = acc + bl[...]


def _layer_tc(agg, cnt, xs, Wl, Wr, bl2d):
  grid = (_NC, _NB)
  return pl.pallas_call(
      _layer_tc_body,
      grid=grid,
      in_specs=[
          pl.BlockSpec((_BR, _H), lambda c, i: (i, 0)),
          pl.BlockSpec((_BR, _H), lambda c, i: (_NB + i, 0)),
          pl.BlockSpec((_BR, _H), lambda c, i: (i, 0)),
          pl.BlockSpec((_BR, _H), lambda c, i: (_NB + i, 0)),
          pl.BlockSpec((_BR, _H), lambda c, i: (i, 0)),
          pl.BlockSpec((_BR, _H), lambda c, i: (_NB + i, 0)),
          pl.BlockSpec((_D, _H), lambda c, i: (0, c)),
          pl.BlockSpec((_D, _H), lambda c, i: (0, c)),
          pl.BlockSpec((1, _H), lambda c, i: (0, c)),
      ],
      out_specs=pl.BlockSpec((_BR, _H), lambda c, i: (c * _NB + i, 0)),
      out_shape=jax.ShapeDtypeStruct((_NC * _N, _H), jnp.float32),
  )(agg, agg, cnt, cnt, xs, xs, Wl, Wr, bl2d)


def _post_tc_body(x0, x1, wp1, bp1, wp2, bp2, out):
  h = jnp.dot(x0[...], wp1[0:_H, :], preferred_element_type=jnp.float32)
  h += jnp.dot(x1[...], wp1[_H:_D, :], preferred_element_type=jnp.float32)
  h = jnp.maximum(h + bp1[...], 0.0)
  out[...] = jnp.dot(h, wp2[...], preferred_element_type=jnp.float32) + bp2[...]


def _post_tc(xs, Wp1, bp1_2d, Wp2, bp2_2d):
  return pl.pallas_call(
      _post_tc_body,
      grid=(_NB,),
      in_specs=[
          pl.BlockSpec((_BR, _H), lambda i: (i, 0)),
          pl.BlockSpec((_BR, _H), lambda i: (_NB + i, 0)),
          pl.BlockSpec((_D, _D), lambda i: (0, 0)),
          pl.BlockSpec((1, _D), lambda i: (0, 0)),
          pl.BlockSpec((_D, _D), lambda i: (0, 0)),
          pl.BlockSpec((1, _D), lambda i: (0, 0)),
      ],
      out_specs=pl.BlockSpec((_BR, _D), lambda i: (i, 0)),
      out_shape=jax.ShapeDtypeStruct((_N, _D), jnp.float32),
  )(xs, xs, Wp1, bp1_2d, Wp2, bp2_2d)


_sc_segsum = _make_sc_segsum()
_sc_count = _make_sc_count()


def kernel(x, edge_attr, edge_index, Wl1, bl1, Wr1, Wl2, bl2, Wr2,
           We1, be1, We2, be2, Wp1, bp1, Wp2, bp2):
  src = edge_index[0]
  dst = edge_index[1]

  src2 = src.reshape(_NS, _EP)
  spad = jnp.full((_NS, _EPP - _EP), _ZROW, jnp.int32)
  sidx = jnp.stack([
      jnp.concatenate([src2, spad], axis=1),
      jnp.concatenate([src2 + _N, spad], axis=1),
  ]).reshape(_NC, _NS, _NCH, _K)
  dst2 = dst.reshape(_NS, _EP)
  dpad = jnp.full((_NS, _EPP - _EP), _N, jnp.int32)
  didx = jnp.concatenate([dst2, dpad], axis=1).reshape(_NS, _NCH, _K)
  didx_cnt = didx.reshape(_NS, _NC, _CC, _K).transpose(1, 0, 2, 3)

  zf = jnp.zeros((_NPAD, _H), jnp.float32)
  ones_h = jnp.ones((_K, _H), jnp.float32)

  xs = x.reshape(_N, _NC, _H).transpose(1, 0, 2).reshape(_NC * _N, _H)

  cnt = _sc_count(didx_cnt, zf, ones_h)
  agg1 = _sc_segsum(xs, sidx, didx, zf)
  x1 = _layer_tc(agg1, cnt, xs, Wl1, Wr1, bl1.reshape(1, _D))

  agg2 = _sc_segsum(x1, sidx, didx, zf)
  x2 = _layer_tc(agg2, cnt, x1, Wl2, Wr2, bl2.reshape(1, _D))

  return _post_tc(x2, Wp1, bp1.reshape(1, _D), Wp2, bp2.reshape(1, _D))


# speedup vs baseline: 1.0111x; 1.0111x over previous
import functools

import jax
import jax.numpy as jnp
from jax import lax
from jax.experimental import pallas as pl
from jax.experimental.pallas import tpu as pltpu
from jax.experimental.pallas import tpu_sc as plsc

_N = 10000
_E = 160000
_D = 256
_H = 128
_NC = 2
_NS = 16
_K = 128
_EP = _E // _NS
_NCH = 80
_EPP = _NCH * _K
_NPAD = 10240
_RPZ = _NPAD // _NS
_RPA = 632
_RPT = _N - 15 * _RPA
_ZROW = 0
_CC = _NCH // _NC
_GW = 16
_GWC = 8
_NB_SEG = 2
_DLY = 1


def _make_sc_segsum():
  mesh = plsc.VectorSubcoreMesh(core_axis_name="c", subcore_axis_name="s",
                                num_cores=_NC, num_subcores=_NS)
  out_type = jax.ShapeDtypeStruct((_NC * _N, _H), jnp.float32)
  scratch = [
      pltpu.VMEM((_GW, _K), jnp.int32),
      pltpu.VMEM((_GW, _K), jnp.int32),
  ] + [pltpu.VMEM((_K, _H), jnp.float32) for _ in range(_NB_SEG)] + [
      pltpu.VMEM_SHARED((_NPAD, _H), jnp.float32),
  ] + [pltpu.SemaphoreType.DMA for _ in range(2 * _NB_SEG)]

  def body(xs, sidx, didx, zf, agg_out, sidx_v, didx_v, *rest):
    bufs = rest[:_NB_SEG]
    agg_sp = rest[_NB_SEG]
    gsems = rest[_NB_SEG + 1:2 * _NB_SEG + 1]
    ssems = rest[2 * _NB_SEG + 1:]
    c = lax.axis_index("c")
    s = lax.axis_index("s")

    pltpu.sync_copy(zf.at[pl.ds(s * _RPZ, _RPZ)],
                    agg_sp.at[pl.ds(s * _RPZ, _RPZ)])
    plsc.subcore_barrier()

    def group(g, carry):
      pltpu.sync_copy(sidx.at[c, s, pl.ds(g * _GW, _GW)], sidx_v)
      pltpu.sync_copy(didx.at[s, pl.ds(g * _GW, _GW)], didx_v)
      gd = {}
      sd = {}

      def scat(j):
        gd[j].wait()
        b = j % _NB_SEG
        sd[j] = pltpu.async_copy(bufs[b], agg_sp.at[didx_v.at[j]],
                                 ssems[b], add=True)

      for j in range(_GW):
        b = j % _NB_SEG
        if j >= _NB_SEG:
          sd[j - _NB_SEG].wait()
        gd[j] = pltpu.async_copy(xs.at[sidx_v.at[j]], bufs[b], gsems[b])
        if j >= _DLY:
          scat(j - _DLY)
      for j in range(_GW - _DLY, _GW):
        scat(j)
      for j in range(_GW - _NB_SEG, _GW):
        sd[j].wait()
      return carry
    lax.fori_loop(0, _NCH // _GW, group, 0)
    plsc.subcore_barrier()

    @pl.when(s < _NS - 1)
    def _():
      pltpu.sync_copy(agg_sp.at[pl.ds(s * _RPA, _RPA)],
                      agg_out.at[pl.ds(c * _N + s * _RPA, _RPA)])

    @pl.when(s == _NS - 1)
    def _():
      base = 15 * _RPA
      pltpu.sync_copy(agg_sp.at[pl.ds(base, _RPT)],
                      agg_out.at[pl.ds(c * _N + base, _RPT)])

  return pl.kernel(body, out_type=out_type, mesh=mesh, scratch_types=scratch)


def _make_sc_count():
  mesh = plsc.VectorSubcoreMesh(core_axis_name="c", subcore_axis_name="s",
                                num_cores=_NC, num_subcores=_NS)
  out_type = jax.ShapeDtypeStruct((_NC * _N, _H), jnp.float32)
  scratch = [
      pltpu.VMEM((_GWC, _K), jnp.int32),
      pltpu.VMEM((_K, _H), jnp.float32),
      pltpu.VMEM_SHARED((_NPAD, _H), jnp.float32),
      pltpu.SemaphoreType.DMA,
  ]

  def body(didx_cnt, zc, ones_h, cnt_out, didx_v, ones_v, cnt_sp, semc):
    c = lax.axis_index("c")
    s = lax.axis_index("s")
    pltpu.sync_copy(ones_h, ones_v)
    pltpu.sync_copy(zc.at[pl.ds(s * _RPZ, _RPZ)],
                    cnt_sp.at[pl.ds(s * _RPZ, _RPZ)])
    plsc.subcore_barrier()

    def group(g, carry):
      pltpu.sync_copy(didx_cnt.at[c, s, pl.ds(g * _GWC, _GWC)], didx_v)
      ds = []
      for j in range(_GWC):
        ds.append(pltpu.async_copy(ones_v, cnt_sp.at[didx_v.at[j]], semc,
                                   add=True))
      for dd in ds:
        dd.wait()
      return carry
    lax.fori_loop(0, _CC // _GWC, group, 0)
    plsc.subcore_barrier()

    @pl.when(s < _NS - 1)
    def _():
      pltpu.sync_copy(cnt_sp.at[pl.ds(s * _RPA, _RPA)],
                      cnt_out.at[pl.ds(c * _N + s * _RPA, _RPA)])

    @pl.when(s == _NS - 1)
    def _():
      base = 15 * _RPA
      pltpu.sync_copy(cnt_sp.at[pl.ds(base, _RPT)],
                      cnt_out.at[pl.ds(c * _N + base, _RPT)])

  return pl.kernel(body, out_type=out_type, mesh=mesh, scratch_types=scratch)


_BR = 1000
_NB = _N // _BR


def _layer_tc_body(agg0, agg1, cnt0, cnt1, x0, x1, wl, wr, bl, out):
  cnt = cnt0[:, 0:1] + cnt1[:, 0:1]
  rcnt = 1.0 / jnp.maximum(cnt, 1.0)
  m0 = agg0[...] * rcnt
  m1 = agg1[...] * rcnt
  acc = jnp.dot(m0, wl[0:_H, :], preferred_element_type=jnp.float32)
  acc += jnp.dot(m1, wl[_H:_D, :], preferred_element_type=jnp.float32)
  acc += jnp.dot(x0[...], wr[0:_H, :], preferred_element_type=jnp.float32)
  acc += jnp.dot(x1[...], wr[_H:_D, :], preferred_element_type=jnp.float32)
  out[...] = acc + bl[...]


def _layer_tc(agg, cnt, xs, Wl, Wr, bl2d):
  grid = (_NC, _NB)
  return pl.pallas_call(
      _layer_tc_body,
      grid=grid,
      in_specs=[
          pl.BlockSpec((_BR, _H), lambda c, i: (i, 0)),
          pl.BlockSpec((_BR, _H), lambda c, i: (_NB + i, 0)),
          pl.BlockSpec((_BR, _H), lambda c, i: (i, 0)),
          pl.BlockSpec((_BR, _H), lambda c, i: (_NB + i, 0)),
          pl.BlockSpec((_BR, _H), lambda c, i: (i, 0)),
          pl.BlockSpec((_BR, _H), lambda c, i: (_NB + i, 0)),
          pl.BlockSpec((_D, _H), lambda c, i: (0, c)),
          pl.BlockSpec((_D, _H), lambda c, i: (0, c)),
          pl.BlockSpec((1, _H), lambda c, i: (0, c)),
      ],
      out_specs=pl.BlockSpec((_BR, _H), lambda c, i: (c * _NB + i, 0)),
      out_shape=jax.ShapeDtypeStruct((_NC * _N, _H), jnp.float32),
  )(agg, agg, cnt, cnt, xs, xs, Wl, Wr, bl2d)


def _post_tc_body(x0, x1, wp1, bp1, wp2, bp2, out):
  h = jnp.dot(x0[...], wp1[0:_H, :], preferred_element_type=jnp.float32)
  h += jnp.dot(x1[...], wp1[_H:_D, :], preferred_element_type=jnp.float32)
  h = jnp.maximum(h + bp1[...], 0.0)
  out[...] = jnp.dot(h, wp2[...], preferred_element_type=jnp.float32) + bp2[...]


def _post_tc(xs, Wp1, bp1_2d, Wp2, bp2_2d):
  return pl.pallas_call(
      _post_tc_body,
      grid=(_NB,),
      in_specs=[
          pl.BlockSpec((_BR, _H), lambda i: (i, 0)),
          pl.BlockSpec((_BR, _H), lambda i: (_NB + i, 0)),
          pl.BlockSpec((_D, _D), lambda i: (0, 0)),
          pl.BlockSpec((1, _D), lambda i: (0, 0)),
          pl.BlockSpec((_D, _D), lambda i: (0, 0)),
          pl.BlockSpec((1, _D), lambda i: (0, 0)),
      ],
      out_specs=pl.BlockSpec((_BR, _D), lambda i: (i, 0)),
      out_shape=jax.ShapeDtypeStruct((_N, _D), jnp.float32),
  )(xs, xs, Wp1, bp1_2d, Wp2, bp2_2d)


_sc_segsum = _make_sc_segsum()
_sc_count = _make_sc_count()


def kernel(x, edge_attr, edge_index, Wl1, bl1, Wr1, Wl2, bl2, Wr2,
           We1, be1, We2, be2, Wp1, bp1, Wp2, bp2):
  src = edge_index[0]
  dst = edge_index[1]

  src2 = src.reshape(_NS, _EP)
  spad = jnp.full((_NS, _EPP - _EP), _ZROW, jnp.int32)
  sidx = jnp.stack([
      jnp.concatenate([src2, spad], axis=1),
      jnp.concatenate([src2 + _N, spad], axis=1),
  ]).reshape(_NC, _NS, _NCH, _K)
  dst2 = dst.reshape(_NS, _EP)
  dpad = jnp.full((_NS, _EPP - _EP), _N, jnp.int32)
  didx = jnp.concatenate([dst2, dpad], axis=1).reshape(_NS, _NCH, _K)
  didx_cnt = didx.reshape(_NS, _NC, _CC, _K).transpose(1, 0, 2, 3)

  zf = jnp.zeros((_NPAD, _H), jnp.float32)
  ones_h = jnp.ones((_K, _H), jnp.float32)

  xs = x.reshape(_N, _NC, _H).transpose(1, 0, 2).reshape(_NC * _N, _H)

  cnt = _sc_count(didx_cnt, zf, ones_h)
  agg1 = _sc_segsum(xs, sidx, didx, zf)
  x1 = _layer_tc(agg1, cnt, xs, Wl1, Wr1, bl1.reshape(1, _D))

  agg2 = _sc_segsum(x1, sidx, didx, zf)
  x2 = _layer_tc(agg2, cnt, x1, Wl2, Wr2, bl2.reshape(1, _D))

  return _post_tc(x2, Wp1, bp1.reshape(1, _D), Wp2, bp2.reshape(1, _D))


# speedup vs baseline: 1.0265x; 1.0152x over previous
import functools

import jax
import jax.numpy as jnp
from jax import lax
from jax.experimental import pallas as pl
from jax.experimental.pallas import tpu as pltpu
from jax.experimental.pallas import tpu_sc as plsc

_N = 10000
_E = 160000
_D = 256
_H = 128
_NC = 2
_NS = 16
_K = 128
_EP = _E // _NS
_NCH = 80
_EPP = _NCH * _K
_NPAD = 10240
_RPZ = _NPAD // _NS
_RPA = 632
_RPT = _N - 15 * _RPA
_ZROW = 0
_CC = _NCH // _NC
_GW = 40
_GWC = 8
_NB_SEG = 2
_DLY = 1


def _make_sc_segsum():
  mesh = plsc.VectorSubcoreMesh(core_axis_name="c", subcore_axis_name="s",
                                num_cores=_NC, num_subcores=_NS)
  out_type = jax.ShapeDtypeStruct((_NC * _N, _H), jnp.float32)
  scratch = [
      pltpu.VMEM((_GW, _K), jnp.int32),
      pltpu.VMEM((_GW, _K), jnp.int32),
  ] + [pltpu.VMEM((_K, _H), jnp.float32) for _ in range(_NB_SEG)] + [
      pltpu.VMEM_SHARED((_NPAD, _H), jnp.float32),
  ] + [pltpu.SemaphoreType.DMA for _ in range(2 * _NB_SEG)]

  def body(xs, sidx, didx, zf, agg_out, sidx_v, didx_v, *rest):
    bufs = rest[:_NB_SEG]
    agg_sp = rest[_NB_SEG]
    gsems = rest[_NB_SEG + 1:2 * _NB_SEG + 1]
    ssems = rest[2 * _NB_SEG + 1:]
    c = lax.axis_index("c")
    s = lax.axis_index("s")

    pltpu.sync_copy(zf.at[pl.ds(s * _RPZ, _RPZ)],
                    agg_sp.at[pl.ds(s * _RPZ, _RPZ)])
    plsc.subcore_barrier()

    def group(g, carry):
      pltpu.sync_copy(sidx.at[c, s, pl.ds(g * _GW, _GW)], sidx_v)
      pltpu.sync_copy(didx.at[s, pl.ds(g * _GW, _GW)], didx_v)
      gd = {}
      sd = {}

      def scat(j):
        gd[j].wait()
        b = j % _NB_SEG
        sd[j] = pltpu.async_copy(bufs[b], agg_sp.at[didx_v.at[j]],
                                 ssems[b], add=True)

      for j in range(_GW):
        b = j % _NB_SEG
        if j >= _NB_SEG:
          sd[j - _NB_SEG].wait()
        gd[j] = pltpu.async_copy(xs.at[sidx_v.at[j]], bufs[b], gsems[b])
        if j >= _DLY:
          scat(j - _DLY)
      for j in range(_GW - _DLY, _GW):
        scat(j)
      for j in range(_GW - _NB_SEG, _GW):
        sd[j].wait()
      return carry
    lax.fori_loop(0, _NCH // _GW, group, 0)
    plsc.subcore_barrier()

    @pl.when(s < _NS - 1)
    def _():
      pltpu.sync_copy(agg_sp.at[pl.ds(s * _RPA, _RPA)],
                      agg_out.at[pl.ds(c * _N + s * _RPA, _RPA)])

    @pl.when(s == _NS - 1)
    def _():
      base = 15 * _RPA
      pltpu.sync_copy(agg_sp.at[pl.ds(base, _RPT)],
                      agg_out.at[pl.ds(c * _N + base, _RPT)])

  return pl.kernel(body, out_type=out_type, mesh=mesh, scratch_types=scratch)


def _make_sc_count():
  mesh = plsc.VectorSubcoreMesh(core_axis_name="c", subcore_axis_name="s",
                                num_cores=_NC, num_subcores=_NS)
  out_type = jax.ShapeDtypeStruct((_NC * _N, _H), jnp.float32)
  scratch = [
      pltpu.VMEM((_GWC, _K), jnp.int32),
      pltpu.VMEM((_K, _H), jnp.float32),
      pltpu.VMEM_SHARED((_NPAD, _H), jnp.float32),
      pltpu.SemaphoreType.DMA,
  ]

  def body(didx_cnt, zc, ones_h, cnt_out, didx_v, ones_v, cnt_sp, semc):
    c = lax.axis_index("c")
    s = lax.axis_index("s")
    pltpu.sync_copy(ones_h, ones_v)
    pltpu.sync_copy(zc.at[pl.ds(s * _RPZ, _RPZ)],
                    cnt_sp.at[pl.ds(s * _RPZ, _RPZ)])
    plsc.subcore_barrier()

    def group(g, carry):
      pltpu.sync_copy(didx_cnt.at[c, s, pl.ds(g * _GWC, _GWC)], didx_v)
      ds = []
      for j in range(_GWC):
        ds.append(pltpu.async_copy(ones_v, cnt_sp.at[didx_v.at[j]], semc,
                                   add=True))
      for dd in ds:
        dd.wait()
      return carry
    lax.fori_loop(0, _CC // _GWC, group, 0)
    plsc.subcore_barrier()

    @pl.when(s < _NS - 1)
    def _():
      pltpu.sync_copy(cnt_sp.at[pl.ds(s * _RPA, _RPA)],
                      cnt_out.at[pl.ds(c * _N + s * _RPA, _RPA)])

    @pl.when(s == _NS - 1)
    def _():
      base = 15 * _RPA
      pltpu.sync_copy(cnt_sp.at[pl.ds(base, _RPT)],
                      cnt_out.at[pl.ds(c * _N + base, _RPT)])

  return pl.kernel(body, out_type=out_type, mesh=mesh, scratch_types=scratch)


_BR = 1000
_NB = _N // _BR


def _layer_tc_body(agg0, agg1, cnt0, cnt1, x0, x1, wl, wr, bl, out):
  cnt = cnt0[:, 0:1] + cnt1[:, 0:1]
  rcnt = 1.0 / jnp.maximum(cnt, 1.0)
  m0 = agg0[...] * rcnt
  m1 = agg1[...] * rcnt
  acc = jnp.dot(m0, wl[0:_H, :], preferred_element_type=jnp.float32)
  acc += jnp.dot(m1, wl[_H:_D, :], preferred_element_type=jnp.float32)
  acc += jnp.dot(x0[...], wr[0:_H, :], preferred_element_type=jnp.float32)
  acc += jnp.dot(x1[...], wr[_H:_D, :], preferred_element_type=jnp.float32)
  out[...] = acc + bl[...]


def _layer_tc(agg, cnt, xs, Wl, Wr, bl2d):
  grid = (_NC, _NB)
  return pl.pallas_call(
      _layer_tc_body,
      grid=grid,
      in_specs=[
          pl.BlockSpec((_BR, _H), lambda c, i: (i, 0)),
          pl.BlockSpec((_BR, _H), lambda c, i: (_NB + i, 0)),
          pl.BlockSpec((_BR, _H), lambda c, i: (i, 0)),
          pl.BlockSpec((_BR, _H), lambda c, i: (_NB + i, 0)),
          pl.BlockSpec((_BR, _H), lambda c, i: (i, 0)),
          pl.BlockSpec((_BR, _H), lambda c, i: (_NB + i, 0)),
          pl.BlockSpec((_D, _H), lambda c, i: (0, c)),
          pl.BlockSpec((_D, _H), lambda c, i: (0, c)),
          pl.BlockSpec((1, _H), lambda c, i: (0, c)),
      ],
      out_specs=pl.BlockSpec((_BR, _H), lambda c, i: (c * _NB + i, 0)),
      out_shape=jax.ShapeDtypeStruct((_NC * _N, _H), jnp.float32),
  )(agg, agg, cnt, cnt, xs, xs, Wl, Wr, bl2d)


def _post_tc_body(x0, x1, wp1, bp1, wp2, bp2, out):
  h = jnp.dot(x0[...], wp1[0:_H, :], preferred_element_type=jnp.float32)
  h += jnp.dot(x1[...], wp1[_H:_D, :], preferred_element_type=jnp.float32)
  h = jnp.maximum(h + bp1[...], 0.0)
  out[...] = jnp.dot(h, wp2[...], preferred_element_type=jnp.float32) + bp2[...]


def _post_tc(xs, Wp1, bp1_2d, Wp2, bp2_2d):
  return pl.pallas_call(
      _post_tc_body,
      grid=(_NB,),
      in_specs=[
          pl.BlockSpec((_BR, _H), lambda i: (i, 0)),
          pl.BlockSpec((_BR, _H), lambda i: (_NB + i, 0)),
          pl.BlockSpec((_D, _D), lambda i: (0, 0)),
          pl.BlockSpec((1, _D), lambda i: (0, 0)),
          pl.BlockSpec((_D, _D), lambda i: (0, 0)),
          pl.BlockSpec((1, _D), lambda i: (0, 0)),
      ],
      out_specs=pl.BlockSpec((_BR, _D), lambda i: (i, 0)),
      out_shape=jax.ShapeDtypeStruct((_N, _D), jnp.float32),
  )(xs, xs, Wp1, bp1_2d, Wp2, bp2_2d)


_sc_segsum = _make_sc_segsum()
_sc_count = _make_sc_count()


def kernel(x, edge_attr, edge_index, Wl1, bl1, Wr1, Wl2, bl2, Wr2,
           We1, be1, We2, be2, Wp1, bp1, Wp2, bp2):
  src = edge_index[0]
  dst = edge_index[1]

  src2 = src.reshape(_NS, _EP)
  spad = jnp.full((_NS, _EPP - _EP), _ZROW, jnp.int32)
  sidx = jnp.stack([
      jnp.concatenate([src2, spad], axis=1),
      jnp.concatenate([src2 + _N, spad], axis=1),
  ]).reshape(_NC, _NS, _NCH, _K)
  dst2 = dst.reshape(_NS, _EP)
  dpad = jnp.full((_NS, _EPP - _EP), _N, jnp.int32)
  didx = jnp.concatenate([dst2, dpad], axis=1).reshape(_NS, _NCH, _K)
  didx_cnt = didx.reshape(_NS, _NC, _CC, _K).transpose(1, 0, 2, 3)

  zf = jnp.zeros((_NPAD, _H), jnp.float32)
  ones_h = jnp.ones((_K, _H), jnp.float32)

  xs = x.reshape(_N, _NC, _H).transpose(1, 0, 2).reshape(_NC * _N, _H)

  cnt = _sc_count(didx_cnt, zf, ones_h)
  agg1 = _sc_segsum(xs, sidx, didx, zf)
  x1 = _layer_tc(agg1, cnt, xs, Wl1, Wr1, bl1.reshape(1, _D))

  agg2 = _sc_segsum(x1, sidx, didx, zf)
  x2 = _layer_tc(agg2, cnt, x1, Wl2, Wr2, bl2.reshape(1, _D))

  return _post_tc(x2, Wp1, bp1.reshape(1, _D), Wp2, bp2.reshape(1, _D))


# speedup vs baseline: 1.0271x; 1.0006x over previous
import functools

import jax
import jax.numpy as jnp
from jax import lax
from jax.experimental import pallas as pl
from jax.experimental.pallas import tpu as pltpu
from jax.experimental.pallas import tpu_sc as plsc

_N = 10000
_E = 160000
_D = 256
_H = 128
_NC = 2
_NS = 16
_K = 128
_EP = _E // _NS
_NCH = 80
_EPP = _NCH * _K
_NPAD = 10240
_RPZ = _NPAD // _NS
_RPA = 632
_RPT = _N - 15 * _RPA
_ZROW = 0
_CC = _NCH // _NC
_GW = 40
_GWC = 40
_NB_SEG = 2
_DLY = 1


def _make_sc_segsum():
  mesh = plsc.VectorSubcoreMesh(core_axis_name="c", subcore_axis_name="s",
                                num_cores=_NC, num_subcores=_NS)
  out_type = jax.ShapeDtypeStruct((_NC * _N, _H), jnp.float32)
  scratch = [
      pltpu.VMEM((_GW, _K), jnp.int32),
      pltpu.VMEM((_GW, _K), jnp.int32),
  ] + [pltpu.VMEM((_K, _H), jnp.float32) for _ in range(_NB_SEG)] + [
      pltpu.VMEM_SHARED((_NPAD, _H), jnp.float32),
  ] + [pltpu.SemaphoreType.DMA for _ in range(2 * _NB_SEG)]

  def body(xs, sidx, didx, zf, agg_out, sidx_v, didx_v, *rest):
    bufs = rest[:_NB_SEG]
    agg_sp = rest[_NB_SEG]
    gsems = rest[_NB_SEG + 1:2 * _NB_SEG + 1]
    ssems = rest[2 * _NB_SEG + 1:]
    c = lax.axis_index("c")
    s = lax.axis_index("s")

    pltpu.sync_copy(zf.at[pl.ds(s * _RPZ, _RPZ)],
                    agg_sp.at[pl.ds(s * _RPZ, _RPZ)])
    plsc.subcore_barrier()

    def group(g, carry):
      pltpu.sync_copy(sidx.at[c, s, pl.ds(g * _GW, _GW)], sidx_v)
      pltpu.sync_copy(didx.at[s, pl.ds(g * _GW, _GW)], didx_v)
      gd = {}
      sd = {}

      def scat(j):
        gd[j].wait()
        b = j % _NB_SEG
        sd[j] = pltpu.async_copy(bufs[b], agg_sp.at[didx_v.at[j]],
                                 ssems[b], add=True)

      for j in range(_GW):
        b = j % _NB_SEG
        if j >= _NB_SEG:
          sd[j - _NB_SEG].wait()
        gd[j] = pltpu.async_copy(xs.at[sidx_v.at[j]], bufs[b], gsems[b])
        if j >= _DLY:
          scat(j - _DLY)
      for j in range(_GW - _DLY, _GW):
        scat(j)
      for j in range(_GW - _NB_SEG, _GW):
        sd[j].wait()
      return carry
    lax.fori_loop(0, _NCH // _GW, group, 0)
    plsc.subcore_barrier()

    @pl.when(s < _NS - 1)
    def _():
      pltpu.sync_copy(agg_sp.at[pl.ds(s * _RPA, _RPA)],
                      agg_out.at[pl.ds(c * _N + s * _RPA, _RPA)])

    @pl.when(s == _NS - 1)
    def _():
      base = 15 * _RPA
      pltpu.sync_copy(agg_sp.at[pl.ds(base, _RPT)],
                      agg_out.at[pl.ds(c * _N + base, _RPT)])

  return pl.kernel(body, out_type=out_type, mesh=mesh, scratch_types=scratch)


def _make_sc_count():
  mesh = plsc.VectorSubcoreMesh(core_axis_name="c", subcore_axis_name="s",
                                num_cores=_NC, num_subcores=_NS)
  out_type = jax.ShapeDtypeStruct((_NC * _N, _H), jnp.float32)
  scratch = [
      pltpu.VMEM((_GWC, _K), jnp.int32),
      pltpu.VMEM((_K, _H), jnp.float32),
      pltpu.VMEM_SHARED((_NPAD, _H), jnp.float32),
      pltpu.SemaphoreType.DMA,
  ]

  def body(didx_cnt, zc, ones_h, cnt_out, didx_v, ones_v, cnt_sp, semc):
    c = lax.axis_index("c")
    s = lax.axis_index("s")
    pltpu.sync_copy(ones_h, ones_v)
    pltpu.sync_copy(zc.at[pl.ds(s * _RPZ, _RPZ)],
                    cnt_sp.at[pl.ds(s * _RPZ, _RPZ)])
    plsc.subcore_barrier()

    def group(g, carry):
      pltpu.sync_copy(didx_cnt.at[c, s, pl.ds(g * _GWC, _GWC)], didx_v)
      ds = []
      for j in range(_GWC):
        ds.append(pltpu.async_copy(ones_v, cnt_sp.at[didx_v.at[j]], semc,
                                   add=True))
      for dd in ds:
        dd.wait()
      return carry
    lax.fori_loop(0, _CC // _GWC, group, 0)
    plsc.subcore_barrier()

    @pl.when(s < _NS - 1)
    def _():
      pltpu.sync_copy(cnt_sp.at[pl.ds(s * _RPA, _RPA)],
                      cnt_out.at[pl.ds(c * _N + s * _RPA, _RPA)])

    @pl.when(s == _NS - 1)
    def _():
      base = 15 * _RPA
      pltpu.sync_copy(cnt_sp.at[pl.ds(base, _RPT)],
                      cnt_out.at[pl.ds(c * _N + base, _RPT)])

  return pl.kernel(body, out_type=out_type, mesh=mesh, scratch_types=scratch)


_BR = 1000
_NB = _N // _BR


def _layer_tc_body(agg0, agg1, cnt0, cnt1, x0, x1, wl, wr, bl, out):
  cnt = cnt0[:, 0:1] + cnt1[:, 0:1]
  rcnt = 1.0 / jnp.maximum(cnt, 1.0)
  m0 = agg0[...] * rcnt
  m1 = agg1[...] * rcnt
  acc = jnp.dot(m0, wl[0:_H, :], preferred_element_type=jnp.float32)
  acc += jnp.dot(m1, wl[_H:_D, :], preferred_element_type=jnp.float32)
  acc += jnp.dot(x0[...], wr[0:_H, :], preferred_element_type=jnp.float32)
  acc += jnp.dot(x1[...], wr[_H:_D, :], preferred_element_type=jnp.float32)
  out[...] = acc + bl[...]


def _layer_tc(agg, cnt, xs, Wl, Wr, bl2d):
  grid = (_NC, _NB)
  return pl.pallas_call(
      _layer_tc_body,
      grid=grid,
      in_specs=[
          pl.BlockSpec((_BR, _H), lambda c, i: (i, 0)),
          pl.BlockSpec((_BR, _H), lambda c, i: (_NB + i, 0)),
          pl.BlockSpec((_BR, _H), lambda c, i: (i, 0)),
          pl.BlockSpec((_BR, _H), lambda c, i: (_NB + i, 0)),
          pl.BlockSpec((_BR, _H), lambda c, i: (i, 0)),
          pl.BlockSpec((_BR, _H), lambda c, i: (_NB + i, 0)),
          pl.BlockSpec((_D, _H), lambda c, i: (0, c)),
          pl.BlockSpec((_D, _H), lambda c, i: (0, c)),
          pl.BlockSpec((1, _H), lambda c, i: (0, c)),
      ],
      out_specs=pl.BlockSpec((_BR, _H), lambda c, i: (c * _NB + i, 0)),
      out_shape=jax.ShapeDtypeStruct((_NC * _N, _H), jnp.float32),
  )(agg, agg, cnt, cnt, xs, xs, Wl, Wr, bl2d)


def _post_tc_body(x0, x1, wp1, bp1, wp2, bp2, out):
  h = jnp.dot(x0[...], wp1[0:_H, :], preferred_element_type=jnp.float32)
  h += jnp.dot(x1[...], wp1[_H:_D, :], preferred_element_type=jnp.float32)
  h = jnp.maximum(h + bp1[...], 0.0)
  out[...] = jnp.dot(h, wp2[...], preferred_element_type=jnp.float32) + bp2[...]


def _post_tc(xs, Wp1, bp1_2d, Wp2, bp2_2d):
  return pl.pallas_call(
      _post_tc_body,
      grid=(_NB,),
      in_specs=[
          pl.BlockSpec((_BR, _H), lambda i: (i, 0)),
          pl.BlockSpec((_BR, _H), lambda i: (_NB + i, 0)),
          pl.BlockSpec((_D, _D), lambda i: (0, 0)),
          pl.BlockSpec((1, _D), lambda i: (0, 0)),
          pl.BlockSpec((_D, _D), lambda i: (0, 0)),
          pl.BlockSpec((1, _D), lambda i: (0, 0)),
      ],
      out_specs=pl.BlockSpec((_BR, _D), lambda i: (i, 0)),
      out_shape=jax.ShapeDtypeStruct((_N, _D), jnp.float32),
  )(xs, xs, Wp1, bp1_2d, Wp2, bp2_2d)


_sc_segsum = _make_sc_segsum()
_sc_count = _make_sc_count()


def kernel(x, edge_attr, edge_index, Wl1, bl1, Wr1, Wl2, bl2, Wr2,
           We1, be1, We2, be2, Wp1, bp1, Wp2, bp2):
  src = edge_index[0]
  dst = edge_index[1]

  src2 = src.reshape(_NS, _EP)
  spad = jnp.full((_NS, _EPP - _EP), _ZROW, jnp.int32)
  sidx = jnp.stack([
      jnp.concatenate([src2, spad], axis=1),
      jnp.concatenate([src2 + _N, spad], axis=1),
  ]).reshape(_NC, _NS, _NCH, _K)
  dst2 = dst.reshape(_NS, _EP)
  dpad = jnp.full((_NS, _EPP - _EP), _N, jnp.int32)
  didx = jnp.concatenate([dst2, dpad], axis=1).reshape(_NS, _NCH, _K)
  didx_cnt = didx.reshape(_NS, _NC, _CC, _K).transpose(1, 0, 2, 3)

  zf = jnp.zeros((_NPAD, _H), jnp.float32)
  ones_h = jnp.ones((_K, _H), jnp.float32)

  xs = x.reshape(_N, _NC, _H).transpose(1, 0, 2).reshape(_NC * _N, _H)

  cnt = _sc_count(didx_cnt, zf, ones_h)
  agg1 = _sc_segsum(xs, sidx, didx, zf)
  x1 = _layer_tc(agg1, cnt, xs, Wl1, Wr1, bl1.reshape(1, _D))

  agg2 = _sc_segsum(x1, sidx, didx, zf)
  x2 = _layer_tc(agg2, cnt, x1, Wl2, Wr2, bl2.reshape(1, _D))

  return _post_tc(x2, Wp1, bp1.reshape(1, _D), Wp2, bp2.reshape(1, _D))
